# Initial kernel scaffold; baseline (speedup 1.0000x reference)
import jax
import jax.numpy as jnp
from jax import lax
from jax.experimental import pallas as pl
from jax.experimental.pallas import tpu as pltpu
from jax.experimental.pallas import tpu_sc as plsc

N = 10000
E = 320000
D = 128

NC = 2
NS = 16
NW = NC * NS

CHUNK = 128
NCHUNKS = E // CHUNK
RPS = N // NS
ZR = 125
DEG_W = 16

_MESH = plsc.VectorSubcoreMesh(core_axis_name="c", subcore_axis_name="s")


def _segsum_body(vals_hbm, src_hbm, dst_hbm, out_hbm, sidx, didx, rows_v, acc_sh, sem):
    cid = lax.axis_index("c")
    sid = lax.axis_index("s")
    wid = sid * NC + cid

    @pl.loop(0, CHUNK)
    def _(i):
        @pl.loop(0, D, step=16)
        def _(j):
            rows_v[i, pl.ds(j, 16)] = jnp.zeros((16,), jnp.float32)

    @pl.loop(0, RPS, step=ZR)
    def _(r):
        pltpu.sync_copy(rows_v.at[pl.ds(0, ZR)],
                        acc_sh.at[pl.ds(sid * RPS + r, ZR)])

    plsc.subcore_barrier()

    @pl.loop(wid, NCHUNKS, step=NW)
    def _(c):
        base = c * CHUNK
        pltpu.sync_copy(src_hbm.at[pl.ds(base, CHUNK)], sidx.at[0])
        pltpu.sync_copy(dst_hbm.at[pl.ds(base, CHUNK)], didx.at[0])
        pltpu.async_copy(vals_hbm.at[sidx.at[0]], rows_v, sem).wait()
        pltpu.sync_copy(rows_v, acc_sh.at[didx.at[0]], add=True)

    plsc.subcore_barrier()
    pltpu.sync_copy(acc_sh.at[pl.ds(sid * RPS, RPS)],
                    out_hbm.at[cid].at[pl.ds(sid * RPS, RPS)])


_segsum = pl.kernel(
    _segsum_body,
    out_type=jax.ShapeDtypeStruct((NC, N, D), jnp.float32),
    mesh=_MESH,
    scratch_types=[
        pltpu.VMEM((1, CHUNK), jnp.int32),
        pltpu.VMEM((1, CHUNK), jnp.int32),
        pltpu.VMEM((CHUNK, D), jnp.float32),
        pltpu.VMEM_SHARED((N, D), jnp.float32),
        pltpu.SemaphoreType.DMA,
    ],
)


def _deg_body(dst_hbm, out_hbm, didx, ones_v, acc_sh):
    cid = lax.axis_index("c")
    sid = lax.axis_index("s")
    wid = sid * NC + cid

    @pl.loop(0, CHUNK)
    def _(i):
        ones_v[i, pl.ds(0, DEG_W)] = jnp.zeros((DEG_W,), jnp.float32)

    @pl.loop(0, RPS, step=ZR)
    def _(r):
        pltpu.sync_copy(ones_v.at[pl.ds(0, ZR)],
                        acc_sh.at[pl.ds(sid * RPS + r, ZR)])

    @pl.loop(0, CHUNK)
    def _(i):
        ones_v[i, pl.ds(0, DEG_W)] = jnp.ones((DEG_W,), jnp.float32)

    plsc.subcore_barrier()

    @pl.loop(wid, NCHUNKS, step=NW)
    def _(c):
        pltpu.sync_copy(dst_hbm.at[pl.ds(c * CHUNK, CHUNK)], didx.at[0])
        pltpu.sync_copy(ones_v, acc_sh.at[didx.at[0]], add=True)

    plsc.subcore_barrier()
    pltpu.sync_copy(acc_sh.at[pl.ds(sid * RPS, RPS)],
                    out_hbm.at[cid].at[pl.ds(sid * RPS, RPS)])


_deg = pl.kernel(
    _deg_body,
    out_type=jax.ShapeDtypeStruct((NC, N, DEG_W), jnp.float32),
    mesh=_MESH,
    scratch_types=[
        pltpu.VMEM((1, CHUNK), jnp.int32),
        pltpu.VMEM((CHUNK, DEG_W), jnp.float32),
        pltpu.VMEM_SHARED((N, DEG_W), jnp.float32),
    ],
)


BN = 2000
NG = N // BN
_PREC = lax.Precision.HIGHEST


def _mm2_body(x_ref, ws_ref, wn_ref, b_ref, os_ref, on_ref):
    x = x_ref[...]
    os_ref[...] = (
        jnp.dot(x, ws_ref[...], precision=_PREC, preferred_element_type=jnp.float32)
        + b_ref[...]
    )
    on_ref[...] = jnp.dot(x, wn_ref[...], precision=_PREC, preferred_element_type=jnp.float32)


def _mm2(x, ws, wn, b):
    return pl.pallas_call(
        _mm2_body,
        grid=(NG,),
        in_specs=[
            pl.BlockSpec((BN, D), lambda i: (i, 0)),
            pl.BlockSpec((D, D), lambda i: (0, 0)),
            pl.BlockSpec((D, D), lambda i: (0, 0)),
            pl.BlockSpec((1, D), lambda i: (0, 0)),
        ],
        out_specs=[
            pl.BlockSpec((BN, D), lambda i: (i, 0)),
            pl.BlockSpec((BN, D), lambda i: (i, 0)),
        ],
        out_shape=[jax.ShapeDtypeStruct((N, D), jnp.float32)] * 2,
    )(x, ws, wn, b.reshape(1, D))


def _combine_body(hws_ref, aggp_ref, degp_ref, ws_ref, wn_ref, b_ref, os_ref, on_ref):
    agg = aggp_ref[0] + aggp_ref[1]
    deg = degp_ref[0, :, 0:1] + degp_ref[1, :, 0:1]
    inv = 1.0 / jnp.maximum(deg, 1.0)
    h = jnp.maximum(hws_ref[...] + agg * inv, 0.0)
    os_ref[...] = (
        jnp.dot(h, ws_ref[...], precision=_PREC, preferred_element_type=jnp.float32)
        + b_ref[...]
    )
    on_ref[...] = jnp.dot(h, wn_ref[...], precision=_PREC, preferred_element_type=jnp.float32)


def _combine_mm2(hws, aggp, degp, ws, wn, b):
    return pl.pallas_call(
        _combine_body,
        grid=(NG,),
        in_specs=[
            pl.BlockSpec((BN, D), lambda i: (i, 0)),
            pl.BlockSpec((NC, BN, D), lambda i: (0, i, 0)),
            pl.BlockSpec((NC, BN, DEG_W), lambda i: (0, i, 0)),
            pl.BlockSpec((D, D), lambda i: (0, 0)),
            pl.BlockSpec((D, D), lambda i: (0, 0)),
            pl.BlockSpec((1, D), lambda i: (0, 0)),
        ],
        out_specs=[
            pl.BlockSpec((BN, D), lambda i: (i, 0)),
            pl.BlockSpec((BN, D), lambda i: (i, 0)),
        ],
        out_shape=[jax.ShapeDtypeStruct((N, D), jnp.float32)] * 2,
    )(hws, aggp, degp, ws, wn, b.reshape(1, D))


def _head_body(hws_ref, aggp_ref, degp_ref, wl0_ref, bl0_ref, wl1_ref, bl1_ref,
               msum_ref, out_ref):
    i = pl.program_id(0)
    agg = aggp_ref[0] + aggp_ref[1]
    deg = degp_ref[0, :, 0:1] + degp_ref[1, :, 0:1]
    inv = 1.0 / jnp.maximum(deg, 1.0)
    h = hws_ref[...] + agg * inv
    t = jnp.maximum(
        jnp.dot(h, wl0_ref[...], precision=_PREC, preferred_element_type=jnp.float32)
        + bl0_ref[...],
        0.0,
    )

    @pl.when(i == 0)
    def _():
        msum_ref[...] = jnp.zeros_like(msum_ref)

    msum_ref[...] += jnp.sum(t, axis=0, keepdims=True)

    @pl.when(i == NG - 1)
    def _():
        out_ref[...] = (
            jnp.dot(msum_ref[...] / N, wl1_ref[...], precision=_PREC,
                    preferred_element_type=jnp.float32)
            + bl1_ref[...]
        )


def _head(hws, aggp, degp, wl0, bl0, wl1, bl1):
    _, out = pl.pallas_call(
        _head_body,
        grid=(NG,),
        in_specs=[
            pl.BlockSpec((BN, D), lambda i: (i, 0)),
            pl.BlockSpec((NC, BN, D), lambda i: (0, i, 0)),
            pl.BlockSpec((NC, BN, DEG_W), lambda i: (0, i, 0)),
            pl.BlockSpec((D, D), lambda i: (0, 0)),
            pl.BlockSpec((1, D), lambda i: (0, 0)),
            pl.BlockSpec((D, 1), lambda i: (0, 0)),
            pl.BlockSpec((1, 1), lambda i: (0, 0)),
        ],
        out_specs=[
            pl.BlockSpec((1, D), lambda i: (0, 0)),
            pl.BlockSpec((1, 1), lambda i: (0, 0)),
        ],
        out_shape=[
            jax.ShapeDtypeStruct((1, D), jnp.float32),
            jax.ShapeDtypeStruct((1, 1), jnp.float32),
        ],
    )(hws, aggp, degp, wl0, bl0.reshape(1, D), wl1, bl1.reshape(1, 1))
    return out


def kernel(x, edge_index, Wself_0, Wneigh_0, bconv_0, Wself_1, Wneigh_1, bconv_1,
           Wself_2, Wneigh_2, bconv_2, Wlin_0, blin_0, Wlin_1, blin_1):
    src = edge_index[0]
    dst = edge_index[1]

    degp = _deg(dst)

    hws, hwn = _mm2(x, Wself_0, Wneigh_0, bconv_0)
    aggp = _segsum(hwn, src, dst)

    hws, hwn = _combine_mm2(hws, aggp, degp, Wself_1, Wneigh_1, bconv_1)
    aggp = _segsum(hwn, src, dst)

    hws, hwn = _combine_mm2(hws, aggp, degp, Wself_2, Wneigh_2, bconv_2)
    aggp = _segsum(hwn, src, dst)

    return _head(hws, aggp, degp, Wlin_0, blin_0, Wlin_1, blin_1)


# speedup vs baseline: 7.4085x; 7.4085x over previous
import jax
import jax.numpy as jnp
from jax import lax
from jax.experimental import pallas as pl
from jax.experimental.pallas import tpu as pltpu
from jax.experimental.pallas import tpu_sc as plsc

N = 10000
E = 320000
D = 128

NC = 2
NS = 16
NW = NC * NS

CHUNK = 128
NCHUNKS = E // CHUNK
RPS = 624
REM = N - NS * RPS
ZC = 104

_MESH = plsc.VectorSubcoreMesh(core_axis_name="c", subcore_axis_name="s",
                               num_cores=NC, num_subcores=NS)


G = 2
NGROUPS = NCHUNKS // G


def _segsum_body(vals_hbm, src_hbm, dst_hbm, out_hbm, sidx, didx, rows_v, acc_sh,
                 gsem, ssem):
    cid = lax.axis_index("c")
    sid = lax.axis_index("s")
    wid = sid * NC + cid
    g0 = wid * NGROUPS // NW
    g1 = (wid + 1) * NGROUPS // NW

    @pl.loop(0, CHUNK)
    def _(i):
        @pl.loop(0, D, step=16)
        def _(j):
            rows_v[0, i, pl.ds(j, 16)] = jnp.zeros((16,), jnp.float32)

    @pl.loop(0, RPS, step=ZC)
    def _(r):
        pltpu.sync_copy(rows_v.at[0].at[pl.ds(0, ZC)],
                        acc_sh.at[pl.ds(sid * RPS + r, ZC)])

    @pl.when(sid == 0)
    def _():
        pltpu.sync_copy(rows_v.at[0].at[pl.ds(0, REM)],
                        acc_sh.at[pl.ds(NS * RPS, REM)])

    plsc.subcore_barrier()

    @pl.loop(g0, g1)
    def _(g):
        base = g * G * CHUNK
        idx_dmas = (
            [pltpu.async_copy(src_hbm.at[pl.ds(base + j * CHUNK, CHUNK)],
                              sidx.at[j], gsem) for j in range(G)]
            + [pltpu.async_copy(dst_hbm.at[pl.ds(base + j * CHUNK, CHUNK)],
                                didx.at[j], gsem) for j in range(G)]
        )
        for d_ in idx_dmas:
            d_.wait()
        gathers = [pltpu.async_copy(vals_hbm.at[sidx.at[j]], rows_v.at[j], gsem)
                   for j in range(G)]
        for d_ in gathers:
            d_.wait()
        scatters = [pltpu.async_copy(rows_v.at[j], acc_sh.at[didx.at[j]], ssem,
                                     add=True)
                    for j in range(G)]
        for d_ in scatters:
            d_.wait()

    plsc.subcore_barrier()
    pltpu.sync_copy(acc_sh.at[pl.ds(sid * RPS, RPS)],
                    out_hbm.at[cid].at[pl.ds(sid * RPS, RPS)])

    @pl.when(sid == 0)
    def _():
        pltpu.sync_copy(acc_sh.at[pl.ds(NS * RPS, REM)],
                        out_hbm.at[cid].at[pl.ds(NS * RPS, REM)])


_segsum = pl.kernel(
    _segsum_body,
    out_type=jax.ShapeDtypeStruct((NC, N, D), jnp.float32),
    mesh=_MESH,
    scratch_types=[
        pltpu.VMEM((G, CHUNK), jnp.int32),
        pltpu.VMEM((G, CHUNK), jnp.int32),
        pltpu.VMEM((G, CHUNK, D), jnp.float32),
        pltpu.VMEM_SHARED((N, D), jnp.float32),
        pltpu.SemaphoreType.DMA,
        pltpu.SemaphoreType.DMA,
    ],
)


def _deg_body(dst_hbm, out_hbm, didx, ones_v, acc_sh, ssem):
    cid = lax.axis_index("c")
    sid = lax.axis_index("s")
    wid = sid * NC + cid

    @pl.loop(0, CHUNK)
    def _(i):
        @pl.loop(0, D, step=16)
        def _(j):
            ones_v[i, pl.ds(j, 16)] = jnp.zeros((16,), jnp.float32)

    @pl.loop(0, RPS, step=ZC)
    def _(r):
        pltpu.sync_copy(ones_v.at[pl.ds(0, ZC)],
                        acc_sh.at[pl.ds(sid * RPS + r, ZC)])

    @pl.when(sid == 0)
    def _():
        pltpu.sync_copy(ones_v.at[pl.ds(0, REM)],
                        acc_sh.at[pl.ds(NS * RPS, REM)])

    @pl.loop(0, CHUNK)
    def _(i):
        ones_v[i, pl.ds(0, 16)] = jnp.ones((16,), jnp.float32)

    plsc.subcore_barrier()

    g0 = wid * NGROUPS // NW
    g1 = (wid + 1) * NGROUPS // NW

    @pl.loop(g0, g1)
    def _(g):
        base = g * G * CHUNK
        idx_dmas = [pltpu.async_copy(dst_hbm.at[pl.ds(base + j * CHUNK, CHUNK)],
                                     didx.at[j], ssem) for j in range(G)]
        for d_ in idx_dmas:
            d_.wait()
        scatters = [pltpu.async_copy(ones_v, acc_sh.at[didx.at[j]], ssem, add=True)
                    for j in range(G)]
        for d_ in scatters:
            d_.wait()

    plsc.subcore_barrier()
    pltpu.sync_copy(acc_sh.at[pl.ds(sid * RPS, RPS)],
                    out_hbm.at[cid].at[pl.ds(sid * RPS, RPS)])

    @pl.when(sid == 0)
    def _():
        pltpu.sync_copy(acc_sh.at[pl.ds(NS * RPS, REM)],
                        out_hbm.at[cid].at[pl.ds(NS * RPS, REM)])


_deg = pl.kernel(
    _deg_body,
    out_type=jax.ShapeDtypeStruct((NC, N, D), jnp.float32),
    mesh=_MESH,
    scratch_types=[
        pltpu.VMEM((G, CHUNK), jnp.int32),
        pltpu.VMEM((CHUNK, D), jnp.float32),
        pltpu.VMEM_SHARED((N, D), jnp.float32),
        pltpu.SemaphoreType.DMA,
    ],
)


BN = 2000
NG = N // BN
_PREC = lax.Precision.HIGHEST


def _mm2_body(x_ref, ws_ref, wn_ref, b_ref, os_ref, on_ref):
    x = x_ref[...]
    os_ref[...] = (
        jnp.dot(x, ws_ref[...], precision=_PREC, preferred_element_type=jnp.float32)
        + b_ref[...]
    )
    on_ref[...] = jnp.dot(x, wn_ref[...], precision=_PREC, preferred_element_type=jnp.float32)


def _mm2(x, ws, wn, b):
    return pl.pallas_call(
        _mm2_body,
        grid=(NG,),
        in_specs=[
            pl.BlockSpec((BN, D), lambda i: (i, 0)),
            pl.BlockSpec((D, D), lambda i: (0, 0)),
            pl.BlockSpec((D, D), lambda i: (0, 0)),
            pl.BlockSpec((1, D), lambda i: (0, 0)),
        ],
        out_specs=[
            pl.BlockSpec((BN, D), lambda i: (i, 0)),
            pl.BlockSpec((BN, D), lambda i: (i, 0)),
        ],
        out_shape=[jax.ShapeDtypeStruct((N, D), jnp.float32)] * 2,
    )(x, ws, wn, b.reshape(1, D))


def _combine_body(hws_ref, aggp_ref, degp_ref, ws_ref, wn_ref, b_ref, os_ref, on_ref):
    agg = aggp_ref[0] + aggp_ref[1]
    deg = degp_ref[0, :, 0:1] + degp_ref[1, :, 0:1]
    inv = 1.0 / jnp.maximum(deg, 1.0)
    h = jnp.maximum(hws_ref[...] + agg * inv, 0.0)
    os_ref[...] = (
        jnp.dot(h, ws_ref[...], precision=_PREC, preferred_element_type=jnp.float32)
        + b_ref[...]
    )
    on_ref[...] = jnp.dot(h, wn_ref[...], precision=_PREC, preferred_element_type=jnp.float32)


def _combine_mm2(hws, aggp, degp, ws, wn, b):
    return pl.pallas_call(
        _combine_body,
        grid=(NG,),
        in_specs=[
            pl.BlockSpec((BN, D), lambda i: (i, 0)),
            pl.BlockSpec((NC, BN, D), lambda i: (0, i, 0)),
            pl.BlockSpec((NC, BN, D), lambda i: (0, i, 0)),
            pl.BlockSpec((D, D), lambda i: (0, 0)),
            pl.BlockSpec((D, D), lambda i: (0, 0)),
            pl.BlockSpec((1, D), lambda i: (0, 0)),
        ],
        out_specs=[
            pl.BlockSpec((BN, D), lambda i: (i, 0)),
            pl.BlockSpec((BN, D), lambda i: (i, 0)),
        ],
        out_shape=[jax.ShapeDtypeStruct((N, D), jnp.float32)] * 2,
    )(hws, aggp, degp, ws, wn, b.reshape(1, D))


def _head_body(hws_ref, aggp_ref, degp_ref, wl0_ref, bl0_ref, wl1_ref, bl1_ref,
               msum_ref, out_ref):
    i = pl.program_id(0)
    agg = aggp_ref[0] + aggp_ref[1]
    deg = degp_ref[0, :, 0:1] + degp_ref[1, :, 0:1]
    inv = 1.0 / jnp.maximum(deg, 1.0)
    h = hws_ref[...] + agg * inv
    t = jnp.maximum(
        jnp.dot(h, wl0_ref[...], precision=_PREC, preferred_element_type=jnp.float32)
        + bl0_ref[...],
        0.0,
    )

    @pl.when(i == 0)
    def _():
        msum_ref[...] = jnp.zeros_like(msum_ref)

    msum_ref[...] += jnp.sum(t, axis=0, keepdims=True)

    @pl.when(i == NG - 1)
    def _():
        out_ref[...] = (
            jnp.dot(msum_ref[...] / N, wl1_ref[...], precision=_PREC,
                    preferred_element_type=jnp.float32)
            + bl1_ref[...]
        )


def _head(hws, aggp, degp, wl0, bl0, wl1, bl1):
    _, out = pl.pallas_call(
        _head_body,
        grid=(NG,),
        in_specs=[
            pl.BlockSpec((BN, D), lambda i: (i, 0)),
            pl.BlockSpec((NC, BN, D), lambda i: (0, i, 0)),
            pl.BlockSpec((NC, BN, D), lambda i: (0, i, 0)),
            pl.BlockSpec((D, D), lambda i: (0, 0)),
            pl.BlockSpec((1, D), lambda i: (0, 0)),
            pl.BlockSpec((D, 1), lambda i: (0, 0)),
            pl.BlockSpec((1, 1), lambda i: (0, 0)),
        ],
        out_specs=[
            pl.BlockSpec((1, D), lambda i: (0, 0)),
            pl.BlockSpec((1, 1), lambda i: (0, 0)),
        ],
        out_shape=[
            jax.ShapeDtypeStruct((1, D), jnp.float32),
            jax.ShapeDtypeStruct((1, 1), jnp.float32),
        ],
    )(hws, aggp, degp, wl0, bl0.reshape(1, D), wl1, bl1.reshape(1, 1))
    return out


def kernel(x, edge_index, Wself_0, Wneigh_0, bconv_0, Wself_1, Wneigh_1, bconv_1,
           Wself_2, Wneigh_2, bconv_2, Wlin_0, blin_0, Wlin_1, blin_1):
    src = edge_index[0]
    dst = edge_index[1]

    degp = _deg(dst)

    hws, hwn = _mm2(x, Wself_0, Wneigh_0, bconv_0)
    aggp = _segsum(hwn, src, dst)

    hws, hwn = _combine_mm2(hws, aggp, degp, Wself_1, Wneigh_1, bconv_1)
    aggp = _segsum(hwn, src, dst)

    hws, hwn = _combine_mm2(hws, aggp, degp, Wself_2, Wneigh_2, bconv_2)
    aggp = _segsum(hwn, src, dst)

    return _head(hws, aggp, degp, Wlin_0, blin_0, Wlin_1, blin_1)
